# baseline (speedup 1.0000x reference)
import functools
import math

import jax
import jax.numpy as jnp
from jax import lax
from jax.experimental import pallas as pl
from jax.experimental.pallas import tpu as pltpu
from jax.experimental.pallas import tpu_sc as plsc

DIM = 768
DS = 16
DCONV = 4
E = 8
DI = 1536
DTR = 48
HID = 4 * DIM
L = 2048

BK1 = 256
TL = 128
BKM = 256
SLOTS = 2 * L + E * BKM
NBLK = SLOTS // BKM

NC = 2
NSUB = 16
NW = NC * NSUB

_f32 = jnp.float32


def _silu(v):
    return v / (1.0 + jnp.exp(-v))


def _rms(v, d):
    ss = jnp.sum(v * v, axis=-1, keepdims=True)
    n = jnp.sqrt(ss)
    return v / jnp.maximum(n, 1e-12) * math.sqrt(d)


def _k2_body(x_ref, win_ref, cwT_ref, misc_ref, alogT_ref,
             wxd_ref, wxbc_ref, wdtp_ref, wout_ref, o_ref,
             dA_s, dBu_s, H_s, h_s, tail_s):
    b = pl.program_id(0)

    @pl.when(b == 0)
    def _():
        h_s[...] = jnp.zeros_like(h_s)

    xn = _rms(x_ref[...], DIM)
    xz = jnp.dot(xn, win_ref[...], preferred_element_type=_f32)
    xi = xz[:, :DI]
    z = xz[:, DI:]
    prev_tail = jnp.where(b == 0, 0.0, tail_s[5:8])
    tail_s[5:8] = xi[TL - 3:TL, :]
    full = jnp.concatenate([prev_tail, xi], axis=0)
    cw = cwT_ref[...]
    xc = (full[0:TL] * cw[0:1] + full[1:TL + 1] * cw[1:2]
          + full[2:TL + 2] * cw[2:3] + full[3:TL + 3] * cw[3:4])
    xc = _silu(xc + misc_ref[0:1])

    dt128 = jnp.dot(xc, wxd_ref[...], preferred_element_type=_f32)
    bc = jnp.dot(xc, wxbc_ref[...], preferred_element_type=_f32)
    v = jnp.dot(dt128, wdtp_ref[...], preferred_element_type=_f32) + misc_ref[1:2]
    delta = jnp.maximum(v, 0.0) + jnp.log(1.0 + jnp.exp(-jnp.abs(v)))

    A = -jnp.exp(alogT_ref[...])
    dA_s[...] = jnp.exp(delta[:, None, :] * A[None, :, :])

    Bm = bc[:, :DS]
    Cm = bc[:, 128:128 + DS]
    u = delta * xc
    dBu_s[...] = u[:, None, :] * Bm[:, :, None]

    def step(t, h):
        for j in range(4):
            h = dA_s[4 * t + j] * h + dBu_s[4 * t + j]
            H_s[4 * t + j] = h
        return h

    h = lax.fori_loop(0, TL // 4, step, h_s[...])
    h_s[...] = h

    ys = jnp.sum(H_s[...] * Cm[:, :, None], axis=1)
    y = ys + xc * misc_ref[2:3]
    y = y * _silu(z)
    o_ref[...] = jnp.dot(y, wout_ref[...], preferred_element_type=_f32) + x_ref[...]


def _k2(x2d, W_in, cwT, misc, alogT, wxd, wxbc, wdtp, W_out):
    full = lambda shape: pl.BlockSpec(shape, lambda b: tuple(0 for _ in shape))
    return pl.pallas_call(
        _k2_body,
        grid=(L // TL,),
        in_specs=[
            pl.BlockSpec((TL, DIM), lambda b: (b, 0)),
            full((DIM, 2 * DI)),
            full((8, DI)),
            full((8, DI)),
            full((DS, DI)),
            full((DI, 128)),
            full((DI, 256)),
            full((128, DI)),
            full((DI, DIM)),
        ],
        out_specs=pl.BlockSpec((TL, DIM), lambda b: (b, 0)),
        out_shape=jax.ShapeDtypeStruct((L, DIM), _f32),
        scratch_shapes=[
            pltpu.VMEM((TL, DS, DI), _f32),
            pltpu.VMEM((TL, DS, DI), _f32),
            pltpu.VMEM((TL, DS, DI), _f32),
            pltpu.VMEM((DS, DI), _f32),
            pltpu.VMEM((8, DI), _f32),
        ],
    )(x2d, W_in, cwT, misc, alogT, wxd, wxbc, wdtp, W_out)


def _k3_body(x2_ref, wg_ref, xn2_ref, p1_ref, p2_ref, g1_ref, g2_ref, eob_ref,
             skip_ref):
    x2 = x2_ref[...]
    xn = _rms(x2, DIM)
    xn2_ref[...] = xn
    logits = jnp.dot(xn, wg_ref[...], preferred_element_type=_f32)
    lane = lax.broadcasted_iota(jnp.int32, (L, 128), 1)
    neg = jnp.float32(-1e30)
    lg = jnp.where(lane < E, logits, neg)
    m1 = jnp.max(lg, axis=1, keepdims=True)
    a1 = jnp.min(jnp.where(lg == m1, lane, 128), axis=1, keepdims=True)
    oh1 = (lane == a1).astype(_f32)
    lg2 = jnp.where(lane == a1, neg, lg)
    m2 = jnp.max(lg2, axis=1, keepdims=True)
    a2 = jnp.min(jnp.where(lg2 == m2, lane, 128), axis=1, keepdims=True)
    oh2 = (lane == a2).astype(_f32)
    g1 = 1.0 / (1.0 + jnp.exp(m2 - m1))
    g2 = 1.0 - g1
    oh = oh1 + oh2

    counts = jnp.sum(oh, axis=0, keepdims=True)
    padc = jnp.ceil(counts / BKM) * BKM
    lt8 = (lax.broadcasted_iota(jnp.int32, (128, 128), 0)
           < lax.broadcasted_iota(jnp.int32, (128, 128), 1)).astype(_f32)
    offpad = jnp.dot(padc, lt8, preferred_element_type=_f32)

    LT = (lax.broadcasted_iota(jnp.int32, (L, L), 1)
          < lax.broadcasted_iota(jnp.int32, (L, L), 0)).astype(jnp.bfloat16)
    cum = jnp.dot(LT, oh.astype(jnp.bfloat16), preferred_element_type=_f32)
    base = cum + offpad
    p1 = jnp.sum(oh1 * base, axis=1, keepdims=True)
    p2 = jnp.sum(oh2 * base, axis=1, keepdims=True)
    p1_ref[...] = jnp.broadcast_to(p1, (L, 128))
    p2_ref[...] = jnp.broadcast_to(p2, (L, 128))
    g1_ref[...] = jnp.broadcast_to(g1, (L, 128))
    g2_ref[...] = jnp.broadcast_to(g2, (L, 128))

    ends = offpad + padc
    bstart = lax.broadcasted_iota(jnp.int32, (64, 128), 0).astype(_f32) * BKM
    lane64 = lax.broadcasted_iota(jnp.int32, (64, 128), 1)
    cnt = jnp.sum(jnp.where((ends <= bstart) & (lane64 < E), 1.0, 0.0),
                  axis=1, keepdims=True)
    eobc = jnp.minimum(cnt, E - 1)
    eob_ref[...] = jnp.broadcast_to(eobc, (64, 128))
    real_end = offpad + counts
    ohb = (lane64.astype(_f32) == eobc).astype(_f32)
    re_b = jnp.sum(ohb * real_end, axis=1, keepdims=True)
    skip_ref[...] = jnp.broadcast_to(
        jnp.where(bstart[:, 0:1] >= re_b, 1.0, 0.0), (64, 128))


def _k3(x2, wg_pad):
    full = lambda shape: pl.BlockSpec(shape, lambda: tuple(0 for _ in shape))
    return pl.pallas_call(
        _k3_body,
        in_specs=[full((L, DIM)), full((DIM, 128))],
        out_specs=[full((L, DIM)), full((L, 128)), full((L, 128)),
                   full((L, 128)), full((L, 128)), full((64, 128)),
                   full((64, 128))],
        out_shape=[
            jax.ShapeDtypeStruct((L, DIM), _f32),
            jax.ShapeDtypeStruct((L, 128), _f32),
            jax.ShapeDtypeStruct((L, 128), _f32),
            jax.ShapeDtypeStruct((L, 128), _f32),
            jax.ShapeDtypeStruct((L, 128), _f32),
            jax.ShapeDtypeStruct((64, 128), _f32),
            jax.ShapeDtypeStruct((64, 128), _f32),
        ],
    )(x2, wg_pad)


def _sc_dispatch(xn2, pos):
    ppw = (2 * L) // NW
    hw = ppw // 2
    mesh = plsc.VectorSubcoreMesh(core_axis_name="c", subcore_axis_name="s")

    @functools.partial(
        pl.kernel,
        out_type=jax.ShapeDtypeStruct((SLOTS, DIM), _f32),
        mesh=mesh,
        scratch_types=[
            pltpu.VMEM((hw,), jnp.int32),
            pltpu.VMEM((hw,), jnp.int32),
            pltpu.VMEM((hw, DIM), _f32),
            pltpu.VMEM((hw, DIM), _f32),
            pltpu.SemaphoreType.DMA,
            pltpu.SemaphoreType.DMA,
        ],
    )
    def _disp(tab_hbm, pos_hbm, out_hbm, idxa, idxb, rowsa, rowsb, seml, sems):
        wid = lax.axis_index("s") * NC + lax.axis_index("c")
        base = wid * ppw
        tokbase = lax.rem(base, L)
        pltpu.sync_copy(pos_hbm.at[pl.ds(base, hw)], idxa)
        pltpu.sync_copy(pos_hbm.at[pl.ds(base + hw, hw)], idxb)
        pltpu.async_copy(tab_hbm.at[pl.ds(tokbase, hw)], rowsa, seml).wait()
        sca = pltpu.async_copy(rowsa, out_hbm.at[idxa], sems)
        ldb = pltpu.async_copy(tab_hbm.at[pl.ds(tokbase + hw, hw)], rowsb, seml)
        sca.wait()
        ldb.wait()
        pltpu.async_copy(rowsb, out_hbm.at[idxb], sems).wait()

    return _disp(xn2, pos)


def _sc_gather2(table, p1, p2):
    rpw = L // NW
    mesh = plsc.VectorSubcoreMesh(core_axis_name="c", subcore_axis_name="s")

    @functools.partial(
        pl.kernel,
        out_type=[
            jax.ShapeDtypeStruct((L, DIM), _f32),
            jax.ShapeDtypeStruct((L, DIM), _f32),
        ],
        mesh=mesh,
        scratch_types=[
            pltpu.VMEM((rpw,), jnp.int32),
            pltpu.VMEM((rpw,), jnp.int32),
            pltpu.VMEM((rpw, DIM), _f32),
            pltpu.VMEM((rpw, DIM), _f32),
            pltpu.SemaphoreType.DMA,
            pltpu.SemaphoreType.DMA,
        ],
    )
    def _gat2(tab_hbm, p1_hbm, p2_hbm, oa_hbm, ob_hbm, idxa, idxb,
              rowsa, rowsb, sema, semb):
        wid = lax.axis_index("s") * NC + lax.axis_index("c")
        base = wid * rpw
        pltpu.sync_copy(p1_hbm.at[pl.ds(base, rpw)], idxa)
        pltpu.sync_copy(p2_hbm.at[pl.ds(base, rpw)], idxb)
        ga = pltpu.async_copy(tab_hbm.at[idxa], rowsa, sema)
        ga.wait()
        gb = pltpu.async_copy(tab_hbm.at[idxb], rowsb, semb)
        pltpu.sync_copy(rowsa, oa_hbm.at[pl.ds(base, rpw)])
        gb.wait()
        pltpu.sync_copy(rowsb, ob_hbm.at[pl.ds(base, rpw)])

    return _gat2(table, p1, p2)


def _k4_body(eob_ref, skip_ref, x_ref, w1_ref, b1_ref, w2_ref, b2_ref, o_ref):
    b = pl.program_id(0)

    @pl.when(skip_ref[b] == 0)
    def _():
        x = x_ref[...]
        h = jnp.dot(x, w1_ref[0], preferred_element_type=_f32) + b1_ref[0]
        h = jax.nn.gelu(h)
        o_ref[...] = jnp.dot(h, w2_ref[0], preferred_element_type=_f32) + b2_ref[0]


def _k4(eob, skip, xsorted, W1, b1r, W2, b2r):
    grid_spec = pltpu.PrefetchScalarGridSpec(
        num_scalar_prefetch=2,
        grid=(NBLK,),
        in_specs=[
            pl.BlockSpec((BKM, DIM), lambda b, eob, skip: (b, 0)),
            pl.BlockSpec((1, DIM, HID), lambda b, eob, skip: (eob[b], 0, 0)),
            pl.BlockSpec((1, 1, HID), lambda b, eob, skip: (eob[b], 0, 0)),
            pl.BlockSpec((1, HID, DIM), lambda b, eob, skip: (eob[b], 0, 0)),
            pl.BlockSpec((1, 1, DIM), lambda b, eob, skip: (eob[b], 0, 0)),
        ],
        out_specs=pl.BlockSpec((BKM, DIM), lambda b, eob, skip: (b, 0)),
    )
    return pl.pallas_call(
        _k4_body,
        grid_spec=grid_spec,
        out_shape=jax.ShapeDtypeStruct((SLOTS, DIM), _f32),
    )(eob, skip, xsorted, W1, b1r, W2, b2r)


def _k5_body(x2_ref, ya_ref, yb_ref, g1_ref, g2_ref, o_ref):
    o_ref[...] = (x2_ref[...]
                  + g1_ref[...][:, 0:1] * ya_ref[...]
                  + g2_ref[...][:, 0:1] * yb_ref[...])


def _k5(x2, yA, yB, g1b, g2b):
    return pl.pallas_call(
        _k5_body,
        grid=(L // BK1,),
        in_specs=[
            pl.BlockSpec((BK1, DIM), lambda b: (b, 0)),
            pl.BlockSpec((BK1, DIM), lambda b: (b, 0)),
            pl.BlockSpec((BK1, DIM), lambda b: (b, 0)),
            pl.BlockSpec((BK1, 128), lambda b: (b, 0)),
            pl.BlockSpec((BK1, 128), lambda b: (b, 0)),
        ],
        out_specs=pl.BlockSpec((BK1, DIM), lambda b: (b, 0)),
        out_shape=jax.ShapeDtypeStruct((L, DIM), _f32),
    )(x2, yA, yB, g1b, g2b)


def kernel(x, W_in, conv_w, conv_b, W_x, W_dt, b_dt, A_log, Dskip, W_out,
           W_gate, W1, b1, W2, b2):
    x2d = x[0]

    cwT = jnp.zeros((8, DI), _f32).at[:DCONV].set(conv_w.T)
    misc = jnp.zeros((8, DI), _f32).at[0].set(conv_b).at[1].set(b_dt).at[2].set(Dskip)
    alogT = A_log.T
    wxd = jnp.zeros((DI, 128), _f32).at[:, :DTR].set(W_x[:, :DTR])
    wxbc = (jnp.zeros((DI, 256), _f32)
            .at[:, :DS].set(W_x[:, DTR:DTR + DS])
            .at[:, 128:128 + DS].set(W_x[:, DTR + DS:DTR + 2 * DS]))
    wdtp = jnp.zeros((128, DI), _f32).at[:DTR].set(W_dt)

    x2 = _k2(x2d, W_in, cwT, misc, alogT, wxd, wxbc, wdtp, W_out)

    wg_pad = jnp.zeros((DIM, 128), _f32).at[:, :E].set(W_gate)
    xn2, p1b, p2b, g1b, g2b, eobb, skipb = _k3(x2, wg_pad)

    p1 = p1b[:, 0].astype(jnp.int32)
    p2 = p2b[:, 0].astype(jnp.int32)
    eob = eobb[:, 0].astype(jnp.int32)[:NBLK]
    skip = skipb[:, 0].astype(jnp.int32)[:NBLK]
    poscat = jnp.concatenate([p1, p2])

    xsorted = _sc_dispatch(xn2, poscat)

    ysorted = _k4(eob, skip, xsorted, W1, b1.reshape(E, 1, HID),
                  W2, b2.reshape(E, 1, DIM))

    yA, yB = _sc_gather2(ysorted, p1, p2)

    out = _k5(x2, yA, yB, g1b, g2b)
    return out[None]


# speedup vs baseline: 1.0047x; 1.0047x over previous
import functools
import math

import jax
import jax.numpy as jnp
from jax import lax
from jax.experimental import pallas as pl
from jax.experimental.pallas import tpu as pltpu
from jax.experimental.pallas import tpu_sc as plsc

DIM = 768
DS = 16
DCONV = 4
E = 8
DI = 1536
DTR = 48
HID = 4 * DIM
L = 2048

BK1 = 256
TL = 128
BKM = 256
SLOTS = 2 * L + E * BKM
NBLK = SLOTS // BKM

NC = 2
NSUB = 16
NW = NC * NSUB

_f32 = jnp.float32


def _silu(v):
    return v / (1.0 + jnp.exp(-v))


def _rms(v, d):
    ss = jnp.sum(v * v, axis=-1, keepdims=True)
    n = jnp.sqrt(ss)
    return v / jnp.maximum(n, 1e-12) * math.sqrt(d)


def _k2_body(x_ref, win_ref, cwT_ref, misc_ref, alogT_ref,
             wxd_ref, wxbc_ref, wdtp_ref, wout_ref, o_ref,
             dA_s, dBu_s, H_s, h_s, tail_s):
    b = pl.program_id(0)

    @pl.when(b == 0)
    def _():
        h_s[...] = jnp.zeros_like(h_s)

    xn = _rms(x_ref[...], DIM)
    xz = jnp.dot(xn, win_ref[...], preferred_element_type=_f32)
    xi = xz[:, :DI]
    z = xz[:, DI:]
    prev_tail = jnp.where(b == 0, 0.0, tail_s[5:8])
    tail_s[5:8] = xi[TL - 3:TL, :]
    full = jnp.concatenate([prev_tail, xi], axis=0)
    cw = cwT_ref[...]
    xc = (full[0:TL] * cw[0:1] + full[1:TL + 1] * cw[1:2]
          + full[2:TL + 2] * cw[2:3] + full[3:TL + 3] * cw[3:4])
    xc = _silu(xc + misc_ref[0:1])

    dt128 = jnp.dot(xc, wxd_ref[...], preferred_element_type=_f32)
    bc = jnp.dot(xc, wxbc_ref[...], preferred_element_type=_f32)
    v = jnp.dot(dt128, wdtp_ref[...], preferred_element_type=_f32) + misc_ref[1:2]
    delta = jnp.maximum(v, 0.0) + jnp.log(1.0 + jnp.exp(-jnp.abs(v)))

    A = -jnp.exp(alogT_ref[...])
    dA_s[...] = jnp.exp(delta[:, None, :] * A[None, :, :])

    Bm = bc[:, :DS]
    Cm = bc[:, 128:128 + DS]
    u = delta * xc
    dBu_s[...] = u[:, None, :] * Bm[:, :, None]

    def step(t, h):
        for j in range(4):
            h = dA_s[4 * t + j] * h + dBu_s[4 * t + j]
            H_s[4 * t + j] = h
        return h

    h = lax.fori_loop(0, TL // 4, step, h_s[...])
    h_s[...] = h

    ys = jnp.sum(H_s[...] * Cm[:, :, None], axis=1)
    y = ys + xc * misc_ref[2:3]
    y = y * _silu(z)
    o_ref[...] = jnp.dot(y, wout_ref[...], preferred_element_type=_f32) + x_ref[...]


def _k2(x2d, W_in, cwT, misc, alogT, wxd, wxbc, wdtp, W_out):
    full = lambda shape: pl.BlockSpec(shape, lambda b: tuple(0 for _ in shape))
    return pl.pallas_call(
        _k2_body,
        grid=(L // TL,),
        in_specs=[
            pl.BlockSpec((TL, DIM), lambda b: (b, 0)),
            full((DIM, 2 * DI)),
            full((8, DI)),
            full((8, DI)),
            full((DS, DI)),
            full((DI, 128)),
            full((DI, 256)),
            full((128, DI)),
            full((DI, DIM)),
        ],
        out_specs=pl.BlockSpec((TL, DIM), lambda b: (b, 0)),
        out_shape=jax.ShapeDtypeStruct((L, DIM), _f32),
        scratch_shapes=[
            pltpu.VMEM((TL, DS, DI), _f32),
            pltpu.VMEM((TL, DS, DI), _f32),
            pltpu.VMEM((TL, DS, DI), _f32),
            pltpu.VMEM((DS, DI), _f32),
            pltpu.VMEM((8, DI), _f32),
        ],
    )(x2d, W_in, cwT, misc, alogT, wxd, wxbc, wdtp, W_out)


def _k3_body(x2_ref, wg_ref, xn2_ref, p1_ref, p2_ref, g1_ref, g2_ref, eob_ref,
             skip_ref):
    x2 = x2_ref[...]
    xn = _rms(x2, DIM)
    xn2_ref[...] = xn
    logits = jnp.dot(xn, wg_ref[...], preferred_element_type=_f32)
    lane = lax.broadcasted_iota(jnp.int32, (L, 128), 1)
    neg = jnp.float32(-1e30)
    lg = jnp.where(lane < E, logits, neg)
    m1 = jnp.max(lg, axis=1, keepdims=True)
    a1 = jnp.min(jnp.where(lg == m1, lane, 128), axis=1, keepdims=True)
    oh1 = (lane == a1).astype(_f32)
    lg2 = jnp.where(lane == a1, neg, lg)
    m2 = jnp.max(lg2, axis=1, keepdims=True)
    a2 = jnp.min(jnp.where(lg2 == m2, lane, 128), axis=1, keepdims=True)
    oh2 = (lane == a2).astype(_f32)
    g1 = 1.0 / (1.0 + jnp.exp(m2 - m1))
    g2 = 1.0 - g1
    oh = oh1 + oh2

    counts = jnp.sum(oh, axis=0, keepdims=True)
    padc = jnp.ceil(counts / BKM) * BKM
    lt8 = (lax.broadcasted_iota(jnp.int32, (128, 128), 0)
           < lax.broadcasted_iota(jnp.int32, (128, 128), 1)).astype(_f32)
    offpad = jnp.dot(padc, lt8, preferred_element_type=_f32)

    LT = (lax.broadcasted_iota(jnp.int32, (L, L), 1)
          < lax.broadcasted_iota(jnp.int32, (L, L), 0)).astype(jnp.bfloat16)
    cum = jnp.dot(LT, oh.astype(jnp.bfloat16), preferred_element_type=_f32)
    base = cum + offpad
    p1 = jnp.sum(oh1 * base, axis=1, keepdims=True)
    p2 = jnp.sum(oh2 * base, axis=1, keepdims=True)
    p1_ref[...] = jnp.broadcast_to(p1, (L, 128))
    p2_ref[...] = jnp.broadcast_to(p2, (L, 128))
    g1_ref[...] = jnp.broadcast_to(g1, (L, 128))
    g2_ref[...] = jnp.broadcast_to(g2, (L, 128))

    ends = offpad + padc
    bstart = lax.broadcasted_iota(jnp.int32, (64, 128), 0).astype(_f32) * BKM
    lane64 = lax.broadcasted_iota(jnp.int32, (64, 128), 1)
    cnt = jnp.sum(jnp.where((ends <= bstart) & (lane64 < E), 1.0, 0.0),
                  axis=1, keepdims=True)
    eobc = jnp.minimum(cnt, E - 1)
    eob_ref[...] = jnp.broadcast_to(eobc, (64, 128))
    real_end = offpad + counts
    ohb = (lane64.astype(_f32) == eobc).astype(_f32)
    re_b = jnp.sum(ohb * real_end, axis=1, keepdims=True)
    skip_ref[...] = jnp.broadcast_to(
        jnp.where(bstart[:, 0:1] >= re_b, 1.0, 0.0), (64, 128))


def _k3(x2, wg_pad):
    full = lambda shape: pl.BlockSpec(shape, lambda: tuple(0 for _ in shape))
    return pl.pallas_call(
        _k3_body,
        in_specs=[full((L, DIM)), full((DIM, 128))],
        out_specs=[full((L, DIM)), full((L, 128)), full((L, 128)),
                   full((L, 128)), full((L, 128)), full((64, 128)),
                   full((64, 128))],
        out_shape=[
            jax.ShapeDtypeStruct((L, DIM), _f32),
            jax.ShapeDtypeStruct((L, 128), _f32),
            jax.ShapeDtypeStruct((L, 128), _f32),
            jax.ShapeDtypeStruct((L, 128), _f32),
            jax.ShapeDtypeStruct((L, 128), _f32),
            jax.ShapeDtypeStruct((64, 128), _f32),
            jax.ShapeDtypeStruct((64, 128), _f32),
        ],
    )(x2, wg_pad)


def _sc_dispatch(xn2, pos):
    ppw = (2 * L) // NW
    mesh = plsc.VectorSubcoreMesh(core_axis_name="c", subcore_axis_name="s")

    @functools.partial(
        pl.kernel,
        out_type=jax.ShapeDtypeStruct((SLOTS, DIM), _f32),
        mesh=mesh,
        scratch_types=[
            pltpu.VMEM((ppw,), jnp.int32),
            pltpu.VMEM((ppw, DIM), _f32),
            pltpu.SemaphoreType.DMA,
        ],
    )
    def _disp(tab_hbm, pos_hbm, out_hbm, idx_v, rows_v, sem):
        wid = lax.axis_index("s") * NC + lax.axis_index("c")
        base = wid * ppw
        tokbase = lax.rem(base, L)
        pltpu.sync_copy(pos_hbm.at[pl.ds(base, ppw)], idx_v)
        pltpu.sync_copy(tab_hbm.at[pl.ds(tokbase, ppw)], rows_v)
        pltpu.async_copy(rows_v, out_hbm.at[idx_v], sem).wait()

    return _disp(xn2, pos)


def _sc_gather2(table, p1, p2):
    rpw = L // NW
    mesh = plsc.VectorSubcoreMesh(core_axis_name="c", subcore_axis_name="s")

    @functools.partial(
        pl.kernel,
        out_type=[
            jax.ShapeDtypeStruct((L, DIM), _f32),
            jax.ShapeDtypeStruct((L, DIM), _f32),
        ],
        mesh=mesh,
        scratch_types=[
            pltpu.VMEM((rpw,), jnp.int32),
            pltpu.VMEM((rpw, DIM), _f32),
            pltpu.SemaphoreType.DMA,
        ],
    )
    def _gat2(tab_hbm, p1_hbm, p2_hbm, oa_hbm, ob_hbm, idx_v, rows_v, sem):
        wid = lax.axis_index("s") * NC + lax.axis_index("c")
        base = wid * rpw
        pltpu.sync_copy(p1_hbm.at[pl.ds(base, rpw)], idx_v)
        pltpu.async_copy(tab_hbm.at[idx_v], rows_v, sem).wait()
        pltpu.sync_copy(rows_v, oa_hbm.at[pl.ds(base, rpw)])
        pltpu.sync_copy(p2_hbm.at[pl.ds(base, rpw)], idx_v)
        pltpu.async_copy(tab_hbm.at[idx_v], rows_v, sem).wait()
        pltpu.sync_copy(rows_v, ob_hbm.at[pl.ds(base, rpw)])

    return _gat2(table, p1, p2)


def _k4_body(eob_ref, skip_ref, x_ref, w1_ref, b1_ref, w2_ref, b2_ref, o_ref):
    b = pl.program_id(0)

    @pl.when(skip_ref[b] == 0)
    def _():
        x = x_ref[...]
        h = jnp.dot(x, w1_ref[0], preferred_element_type=_f32) + b1_ref[0]
        h = jax.nn.gelu(h)
        o_ref[...] = jnp.dot(h, w2_ref[0], preferred_element_type=_f32) + b2_ref[0]


def _k4(eob, skip, xsorted, W1, b1r, W2, b2r):
    grid_spec = pltpu.PrefetchScalarGridSpec(
        num_scalar_prefetch=2,
        grid=(NBLK,),
        in_specs=[
            pl.BlockSpec((BKM, DIM), lambda b, eob, skip: (b, 0)),
            pl.BlockSpec((1, DIM, HID), lambda b, eob, skip: (eob[b], 0, 0)),
            pl.BlockSpec((1, 1, HID), lambda b, eob, skip: (eob[b], 0, 0)),
            pl.BlockSpec((1, HID, DIM), lambda b, eob, skip: (eob[b], 0, 0)),
            pl.BlockSpec((1, 1, DIM), lambda b, eob, skip: (eob[b], 0, 0)),
        ],
        out_specs=pl.BlockSpec((BKM, DIM), lambda b, eob, skip: (b, 0)),
    )
    return pl.pallas_call(
        _k4_body,
        grid_spec=grid_spec,
        out_shape=jax.ShapeDtypeStruct((SLOTS, DIM), _f32),
    )(eob, skip, xsorted, W1, b1r, W2, b2r)


def _k5_body(x2_ref, ya_ref, yb_ref, g1_ref, g2_ref, o_ref):
    o_ref[...] = (x2_ref[...]
                  + g1_ref[...][:, 0:1] * ya_ref[...]
                  + g2_ref[...][:, 0:1] * yb_ref[...])


def _k5(x2, yA, yB, g1b, g2b):
    return pl.pallas_call(
        _k5_body,
        grid=(L // BK1,),
        in_specs=[
            pl.BlockSpec((BK1, DIM), lambda b: (b, 0)),
            pl.BlockSpec((BK1, DIM), lambda b: (b, 0)),
            pl.BlockSpec((BK1, DIM), lambda b: (b, 0)),
            pl.BlockSpec((BK1, 128), lambda b: (b, 0)),
            pl.BlockSpec((BK1, 128), lambda b: (b, 0)),
        ],
        out_specs=pl.BlockSpec((BK1, DIM), lambda b: (b, 0)),
        out_shape=jax.ShapeDtypeStruct((L, DIM), _f32),
    )(x2, yA, yB, g1b, g2b)


def kernel(x, W_in, conv_w, conv_b, W_x, W_dt, b_dt, A_log, Dskip, W_out,
           W_gate, W1, b1, W2, b2):
    x2d = x[0]

    cwT = jnp.zeros((8, DI), _f32).at[:DCONV].set(conv_w.T)
    misc = jnp.zeros((8, DI), _f32).at[0].set(conv_b).at[1].set(b_dt).at[2].set(Dskip)
    alogT = A_log.T
    wxd = jnp.zeros((DI, 128), _f32).at[:, :DTR].set(W_x[:, :DTR])
    wxbc = (jnp.zeros((DI, 256), _f32)
            .at[:, :DS].set(W_x[:, DTR:DTR + DS])
            .at[:, 128:128 + DS].set(W_x[:, DTR + DS:DTR + 2 * DS]))
    wdtp = jnp.zeros((128, DI), _f32).at[:DTR].set(W_dt)

    x2 = _k2(x2d, W_in, cwT, misc, alogT, wxd, wxbc, wdtp, W_out)

    wg_pad = jnp.zeros((DIM, 128), _f32).at[:, :E].set(W_gate)
    xn2, p1b, p2b, g1b, g2b, eobb, skipb = _k3(x2, wg_pad)

    p1 = p1b[:, 0].astype(jnp.int32)
    p2 = p2b[:, 0].astype(jnp.int32)
    eob = eobb[:, 0].astype(jnp.int32)[:NBLK]
    skip = skipb[:, 0].astype(jnp.int32)[:NBLK]
    poscat = jnp.concatenate([p1, p2])

    xsorted = _sc_dispatch(xn2, poscat)

    ysorted = _k4(eob, skip, xsorted, W1, b1.reshape(E, 1, HID),
                  W2, b2.reshape(E, 1, DIM))

    yA, yB = _sc_gather2(ysorted, p1, p2)

    out = _k5(x2, yA, yB, g1b, g2b)
    return out[None]
